# baseline (speedup 1.0000x reference)
import functools

import jax
import jax.numpy as jnp
from jax import lax
from jax.experimental import pallas as pl
from jax.experimental.pallas import tpu as pltpu
from jax.experimental.pallas import tpu_sc as plsc

_NC = 2
_NS = 16
_NW = _NC * _NS
_D = 128
_SEQ = 16384
_BPW = _SEQ // _NW
_CHUNK = 128
_NCHUNK = _BPW // _CHUNK


_STEP = 256
_NSTEP = 3 * _BPW // _STEP
_CPS = _STEP // _CHUNK
_NBUF = 3


def _gather3(idx, w0, w1, w2):
    mesh = plsc.VectorSubcoreMesh(core_axis_name="c", subcore_axis_name="s")
    out = jax.ShapeDtypeStruct((_SEQ, _D), jnp.float32)
    buf_t = pltpu.VMEM((_STEP, _D), jnp.float32)

    @functools.partial(
        pl.kernel,
        out_type=(out,) * 6,
        mesh=mesh,
        scratch_types=[
            pltpu.VMEM((_NCHUNK, _CHUNK), jnp.int32),
            (buf_t,) * _NBUF,
            pltpu.SemaphoreType.DMA,
            pltpu.SemaphoreType.DMA,
            pltpu.SemaphoreType.DMA,
        ],
    )
    def k(idx_hbm, w0_hbm, w1_hbm, w2_hbm,
          o0_hbm, o1_hbm, o2_hbm, o3_hbm, o4_hbm, o5_hbm,
          idx_v, bufs, gsem, osem, dsem):
        wid = lax.axis_index("s") * _NC + lax.axis_index("c")
        base = wid * _BPW
        pltpu.sync_copy(idx_hbm.at[wid], idx_v)
        prim = (o0_hbm, o1_hbm, o2_hbm)
        dup = (o3_hbm, o4_hbm, o5_hbm)
        tabs = (w0_hbm, w1_hbm, w2_hbm)

        def fire_gather(s):
            t, h = divmod(s, _BPW // _STEP)
            return [
                pltpu.async_copy(
                    tabs[t].at[idx_v.at[h * _CPS + j]],
                    bufs[s % _NBUF].at[pl.ds(j * _CHUNK, _CHUNK)],
                    gsem,
                )
                for j in range(_CPS)
            ]

        def fire_out(s):
            t, h = divmod(s, _BPW // _STEP)
            return pltpu.async_copy(
                bufs[s % _NBUF],
                prim[t].at[pl.ds(base + h * _STEP, _STEP)],
                osem,
            )

        def fire_dup(s):
            t, h = divmod(s, _BPW // _STEP)
            sl = pl.ds(base + h * _STEP, _STEP)
            return pltpu.async_copy(prim[t].at[sl], dup[t].at[sl], dsem)

        gd, od, dd = {}, {}, {}
        for s in range(_NSTEP):
            if s >= _NBUF:
                od[s - _NBUF].wait()
                dd[s - _NBUF] = fire_dup(s - _NBUF)
            gd[s] = fire_gather(s)
            if s >= 1:
                for d in gd[s - 1]:
                    d.wait()
                od[s - 1] = fire_out(s - 1)
        for d in gd[_NSTEP - 1]:
            d.wait()
        od[_NSTEP - 1] = fire_out(_NSTEP - 1)
        for s in range(_NSTEP - _NBUF, _NSTEP):
            od[s].wait()
            dd[s] = fire_dup(s)
        for s in range(_NSTEP):
            dd[s].wait()

    return k(idx, w0, w1, w2)


def kernel(input_seq, W0, W1, W2):
    idx = input_seq.reshape(_NW, _NCHUNK, _CHUNK)
    o0, o1, o2, o3, o4, o5 = _gather3(idx, W0, W1, W2)
    return (o0, o1, o2, o3, o4, o5)


# speedup vs baseline: 16.7771x; 16.7771x over previous
import functools

import jax
import jax.numpy as jnp
from jax import lax
from jax.experimental import pallas as pl
from jax.experimental.pallas import tpu as pltpu
from jax.experimental.pallas import tpu_sc as plsc

_NC = 2
_NS = 16
_NW = _NC * _NS
_D = 128
_SEQ = 16384
_BPW = _SEQ // _NW
_CHUNK = 128
_NCHUNK = _BPW // _CHUNK


_STEP = 128
_NSTEP = 3 * _BPW // _STEP
_CPS = _STEP // _CHUNK
_NBUF = 6


def _gather3(idx, w0, w1, w2):
    mesh = plsc.VectorSubcoreMesh(core_axis_name="c", subcore_axis_name="s")
    out = jax.ShapeDtypeStruct((_SEQ, _D), jnp.float32)
    buf_t = pltpu.VMEM((_STEP, _D), jnp.float32)

    @functools.partial(
        pl.kernel,
        out_type=(out,) * 6,
        mesh=mesh,
        scratch_types=[
            pltpu.VMEM((_NCHUNK, _CHUNK), jnp.int32),
            (buf_t,) * _NBUF,
            pltpu.SemaphoreType.DMA,
            pltpu.SemaphoreType.DMA,
        ],
    )
    def k(idx_hbm, w0_hbm, w1_hbm, w2_hbm,
          o0_hbm, o1_hbm, o2_hbm, o3_hbm, o4_hbm, o5_hbm,
          idx_v, bufs, gsem, osem):
        wid = lax.axis_index("s") * _NC + lax.axis_index("c")
        base = wid * _BPW
        pltpu.sync_copy(idx_hbm.at[wid], idx_v)
        outs = ((o0_hbm, o3_hbm), (o1_hbm, o4_hbm), (o2_hbm, o5_hbm))
        tabs = (w0_hbm, w1_hbm, w2_hbm)

        def fire_gather(s):
            t, h = divmod(s, _BPW // _STEP)
            return [
                pltpu.async_copy(
                    tabs[t].at[idx_v.at[h * _CPS + j]],
                    bufs[s % _NBUF].at[pl.ds(j * _CHUNK, _CHUNK)],
                    gsem,
                )
                for j in range(_CPS)
            ]

        def fire_out(s):
            t, h = divmod(s, _BPW // _STEP)
            return [
                pltpu.async_copy(
                    bufs[s % _NBUF],
                    o.at[pl.ds(base + h * _STEP, _STEP)],
                    osem,
                )
                for o in outs[t]
            ]

        gd, od = {}, {}
        for s in range(_NSTEP):
            if s >= _NBUF:
                for d in od[s - _NBUF]:
                    d.wait()
            gd[s] = fire_gather(s)
            if s >= 1:
                for d in gd[s - 1]:
                    d.wait()
                od[s - 1] = fire_out(s - 1)
        for d in gd[_NSTEP - 1]:
            d.wait()
        od[_NSTEP - 1] = fire_out(_NSTEP - 1)
        for s in range(_NSTEP - _NBUF, _NSTEP):
            for d in od[s]:
                d.wait()

    return k(idx, w0, w1, w2)


def kernel(input_seq, W0, W1, W2):
    idx = input_seq.reshape(_NW, _NCHUNK, _CHUNK)
    o0, o1, o2, o3, o4, o5 = _gather3(idx, W0, W1, W2)
    return (o0, o1, o2, o3, o4, o5)


# speedup vs baseline: 16.9098x; 1.0079x over previous
import functools

import jax
import jax.numpy as jnp
from jax import lax
from jax.experimental import pallas as pl
from jax.experimental.pallas import tpu as pltpu
from jax.experimental.pallas import tpu_sc as plsc

_NC = 2
_NS = 16
_NW = _NC * _NS
_D = 128
_SEQ = 16384
_BPW = _SEQ // _NW
_CHUNK = 128
_NCHUNK = _BPW // _CHUNK


_STEP = 256
_NSTEP = 3 * _BPW // _STEP
_CPS = _STEP // _CHUNK
_NBUF = 2


def _gather3(idx, w0, w1, w2):
    mesh = plsc.VectorSubcoreMesh(core_axis_name="c", subcore_axis_name="s")
    out = jax.ShapeDtypeStruct((_SEQ, _D), jnp.float32)
    buf_t = pltpu.VMEM((_STEP, _D), jnp.float32)

    @functools.partial(
        pl.kernel,
        out_type=(out,) * 6,
        mesh=mesh,
        scratch_types=[
            pltpu.VMEM((_NCHUNK, _CHUNK), jnp.int32),
            (buf_t,) * _NBUF,
            pltpu.SemaphoreType.DMA,
            pltpu.SemaphoreType.DMA,
        ],
    )
    def k(idx_hbm, w0_hbm, w1_hbm, w2_hbm,
          o0_hbm, o1_hbm, o2_hbm, o3_hbm, o4_hbm, o5_hbm,
          idx_v, bufs, gsem, osem):
        wid = lax.axis_index("s") * _NC + lax.axis_index("c")
        base = wid * _BPW
        pltpu.sync_copy(idx_hbm.at[wid], idx_v)
        outs = ((o0_hbm, o3_hbm), (o1_hbm, o4_hbm), (o2_hbm, o5_hbm))
        tabs = (w0_hbm, w1_hbm, w2_hbm)

        def fire_gather(s):
            t, h = divmod(s, _BPW // _STEP)
            return [
                pltpu.async_copy(
                    tabs[t].at[idx_v.at[h * _CPS + j]],
                    bufs[s % _NBUF].at[pl.ds(j * _CHUNK, _CHUNK)],
                    gsem,
                )
                for j in range(_CPS)
            ]

        def fire_out(s):
            t, h = divmod(s, _BPW // _STEP)
            return [
                pltpu.async_copy(
                    bufs[s % _NBUF],
                    o.at[pl.ds(base + h * _STEP, _STEP)],
                    osem,
                )
                for o in outs[t]
            ]

        gd, od = {}, {}
        for s in range(_NSTEP):
            if s >= _NBUF:
                for d in od[s - _NBUF]:
                    d.wait()
            gd[s] = fire_gather(s)
            if s >= 1:
                for d in gd[s - 1]:
                    d.wait()
                od[s - 1] = fire_out(s - 1)
        for d in gd[_NSTEP - 1]:
            d.wait()
        od[_NSTEP - 1] = fire_out(_NSTEP - 1)
        for s in range(_NSTEP - _NBUF, _NSTEP):
            for d in od[s]:
                d.wait()

    return k(idx, w0, w1, w2)


def kernel(input_seq, W0, W1, W2):
    idx = input_seq.reshape(_NW, _NCHUNK, _CHUNK)
    o0, o1, o2, o3, o4, o5 = _gather3(idx, W0, W1, W2)
    return (o0, o1, o2, o3, o4, o5)


# speedup vs baseline: 17.0476x; 1.0081x over previous
import functools

import jax
import jax.numpy as jnp
from jax import lax
from jax.experimental import pallas as pl
from jax.experimental.pallas import tpu as pltpu
from jax.experimental.pallas import tpu_sc as plsc

_NC = 2
_NS = 16
_NW = _NC * _NS
_D = 128
_SEQ = 16384
_BPW = _SEQ // _NW
_CHUNK = 128
_NCHUNK = _BPW // _CHUNK


_STEP = 256
_NSTEP = 3 * _BPW // _STEP
_CPS = _STEP // _CHUNK
_NBUF = 3


def _gather3(idx, w0, w1, w2):
    mesh = plsc.VectorSubcoreMesh(core_axis_name="c", subcore_axis_name="s")
    out = jax.ShapeDtypeStruct((_SEQ, _D), jnp.float32)
    buf_t = pltpu.VMEM((_STEP, _D), jnp.float32)

    @functools.partial(
        pl.kernel,
        out_type=(out,) * 6,
        mesh=mesh,
        scratch_types=[
            pltpu.VMEM((_NCHUNK, _CHUNK), jnp.int32),
            (buf_t,) * _NBUF,
            pltpu.SemaphoreType.DMA,
            pltpu.SemaphoreType.DMA,
        ],
    )
    def k(idx_hbm, w0_hbm, w1_hbm, w2_hbm,
          o0_hbm, o1_hbm, o2_hbm, o3_hbm, o4_hbm, o5_hbm,
          idx_v, bufs, gsem, osem):
        wid = lax.axis_index("s") * _NC + lax.axis_index("c")
        base = wid * _BPW
        pltpu.sync_copy(idx_hbm.at[wid], idx_v)
        outs = ((o0_hbm, o3_hbm), (o1_hbm, o4_hbm), (o2_hbm, o5_hbm))
        tabs = (w0_hbm, w1_hbm, w2_hbm)

        def fire_gather(s):
            t, h = divmod(s, _BPW // _STEP)
            return [
                pltpu.async_copy(
                    tabs[t].at[idx_v.at[h * _CPS + j]],
                    bufs[s % _NBUF].at[pl.ds(j * _CHUNK, _CHUNK)],
                    gsem,
                )
                for j in range(_CPS)
            ]

        def fire_out(s):
            t, h = divmod(s, _BPW // _STEP)
            return [
                pltpu.async_copy(
                    bufs[s % _NBUF],
                    o.at[pl.ds(base + h * _STEP, _STEP)],
                    osem,
                )
                for o in outs[t]
            ]

        gd, od = {}, {}
        for s in range(_NSTEP):
            if s >= _NBUF:
                for d in od[s - _NBUF]:
                    d.wait()
            gd[s] = fire_gather(s)
            if s >= 1:
                for d in gd[s - 1]:
                    d.wait()
                od[s - 1] = fire_out(s - 1)
        for d in gd[_NSTEP - 1]:
            d.wait()
        od[_NSTEP - 1] = fire_out(_NSTEP - 1)
        for s in range(_NSTEP - _NBUF, _NSTEP):
            for d in od[s]:
                d.wait()

    return k(idx, w0, w1, w2)


def kernel(input_seq, W0, W1, W2):
    idx = input_seq.reshape(_NW, _NCHUNK, _CHUNK)
    o0, o1, o2, o3, o4, o5 = _gather3(idx, W0, W1, W2)
    return (o0, o1, o2, o3, o4, o5)
